# baseline (speedup 1.0000x reference)
import functools

import jax
import jax.numpy as jnp
from jax import lax
from jax.experimental import pallas as pl
from jax.experimental.pallas import tpu as pltpu
from jax.experimental.pallas import tpu_sc as plsc

N = 10000
E = 320000
D = 128

NC = 2
NS = 16
B = 128

NB = 80
NBT = NC * NS * NB
CH = 8
NB0 = 160
NB1 = 0
EWK = NB * B
EP = NC * NS * EWK
NP = 10112
RPS = NP // NS

def _make_mesh():
    return plsc.VectorSubcoreMesh(
        core_axis_name="c", subcore_axis_name="s", num_cores=NC, num_subcores=NS
    )


def _k1_body(row3_hbm, deg_out, deg_sp, idx_all, ones_v, zero_v, sem):
    cid = lax.axis_index("c")
    sid = lax.axis_index("s")
    w = cid * NS + sid

    for i in range(B // 16):
        ones_v[pl.ds(i * 16, 16)] = jnp.full((16,), 1.0, jnp.float32)
    for i in range(RPS // 16):
        zero_v[pl.ds(i * 16, 16)] = jnp.zeros((16,), jnp.float32)
    pltpu.sync_copy(zero_v, deg_sp.at[pl.ds(sid * RPS, RPS)])
    pltpu.sync_copy(row3_hbm.at[pl.ds(w * NB, NB)], idx_all)
    plsc.subcore_barrier()

    def body(j, carry):
        pltpu.sync_copy(ones_v, deg_sp.at[idx_all.at[j]], add=True)
        return carry

    lax.fori_loop(0, NB, body, 0)
    plsc.subcore_barrier()
    pltpu.sync_copy(deg_sp.at[pl.ds(sid * RPS, RPS)], zero_v)
    pltpu.sync_copy(zero_v, deg_out.at[pl.ds(cid * NP + sid * RPS, RPS)])


@functools.cache
def _k1_deg():
    return pl.kernel(
        _k1_body,
        mesh=_make_mesh(),
        out_type=jax.ShapeDtypeStruct((NC * NP,), jnp.float32),
        scratch_types=[
            pltpu.VMEM_SHARED((NP,), jnp.float32),
            pltpu.VMEM((NB, B), jnp.int32),
            pltpu.VMEM((B,), jnp.float32),
            pltpu.VMEM((RPS,), jnp.float32),
            pltpu.SemaphoreType.DMA,
        ],
    )


def _k3_body(col3_hbm, row3_hbm, xs_hbm, h_out, h_sp, idx_c, idx_r, msg0, msg1, sg0, sg1):
    cid = lax.axis_index("c")
    sid = lax.axis_index("s")
    w = cid * NS + sid

    def zrow(r, carry):
        for i in range(D // 16):
            msg0[r, pl.ds(i * 16, 16)] = jnp.zeros((16,), jnp.float32)
        return carry

    lax.fori_loop(0, B, zrow, 0)
    r0 = sid * RPS
    rem = RPS % B
    for k in range(RPS // B):
        pltpu.sync_copy(msg0, h_sp.at[pl.ds(r0 + k * B, B), :])
    pltpu.sync_copy(
        msg0.at[pl.ds(0, rem), :], h_sp.at[pl.ds(r0 + (RPS // B) * B, rem), :]
    )
    plsc.subcore_barrier()

    bt0 = lax.select(cid == 0, sid * NB0, NS * NB0 + sid * NB1)
    nch = lax.select(cid == 0, NB0 // CH, NB1 // CH)

    def chunk(c, carry):
        pltpu.sync_copy(col3_hbm.at[pl.ds(bt0 + c * CH, CH)], idx_c)
        pltpu.sync_copy(row3_hbm.at[pl.ds(bt0 + c * CH, CH)], idx_r)
        bufs = (msg0, msg1)
        sems = (sg0, sg1)
        descs = [None] * CH
        descs[0] = pltpu.async_copy(xs_hbm.at[idx_c.at[0]], msg0, sg0)
        for j in range(CH):
            if j + 1 < CH:
                descs[j + 1] = pltpu.async_copy(
                    xs_hbm.at[idx_c.at[j + 1]], bufs[(j + 1) % 2], sems[(j + 1) % 2]
                )
            descs[j].wait()
            pltpu.sync_copy(bufs[j % 2], h_sp.at[idx_r.at[j]], add=True)
        return carry

    lax.fori_loop(0, nch, chunk, 0)
    plsc.subcore_barrier()

    @pl.when(cid == 0)
    def _copy_out():
        for k in range(RPS // B):
            pltpu.sync_copy(h_sp.at[pl.ds(r0 + k * B, B), :], msg0)
            pltpu.sync_copy(msg0, h_out.at[pl.ds(r0 + k * B, B), :])
        pltpu.sync_copy(
            h_sp.at[pl.ds(r0 + (RPS // B) * B, rem), :], msg0.at[pl.ds(0, rem), :]
        )
        pltpu.sync_copy(
            msg0.at[pl.ds(0, rem), :], h_out.at[pl.ds(r0 + (RPS // B) * B, rem), :]
        )


@functools.cache
def _k3_spmm():
    return pl.kernel(
        _k3_body,
        mesh=_make_mesh(),
        out_type=jax.ShapeDtypeStruct((NP, D), jnp.float32),
        scratch_types=[
            pltpu.VMEM_SHARED((NP, D), jnp.float32),
            pltpu.VMEM((CH, B), jnp.int32),
            pltpu.VMEM((CH, B), jnp.int32),
            pltpu.VMEM((B, D), jnp.float32),
            pltpu.VMEM((B, D), jnp.float32),
            pltpu.SemaphoreType.DMA,
            pltpu.SemaphoreType.DMA,
        ],
    )


R = 1000
GRID = N // R


def _k2_body(dp_ref, x_ref, xs_ref):
    d = dp_ref[0] + dp_ref[1] + 1.0
    dis = lax.rsqrt(d + 1e-12)
    xs_ref[...] = x_ref[...] * dis


def _k2_prescale(degp2, x):
    return pl.pallas_call(
        _k2_body,
        grid=(GRID,),
        in_specs=[
            pl.BlockSpec((NC, R, 1), lambda i: (0, i, 0)),
            pl.BlockSpec((R, D), lambda i: (i, 0)),
        ],
        out_specs=pl.BlockSpec((R, D), lambda i: (i, 0)),
        out_shape=jax.ShapeDtypeStruct((N, D), jnp.float32),
    )(degp2, x)


def _k4_body(ta_ref, xs_ref, dp_ref, x_ref, wt_ref, b_ref, g_ref, bt_ref, o_ref):
    d = dp_ref[0] + dp_ref[1] + 1.0
    dis = lax.rsqrt(d + 1e-12)
    h = (ta_ref[...] + xs_ref[...]) * dis
    y = jnp.dot(h, wt_ref[...], preferred_element_type=jnp.float32) + b_ref[...]
    y = 0.5 * y * (1.0 + lax.erf(y * 0.7071067811865475))
    mu = jnp.mean(y, axis=-1, keepdims=True)
    yc = y - mu
    var = jnp.mean(yc * yc, axis=-1, keepdims=True)
    y = yc * lax.rsqrt(var + 1e-5) * g_ref[...] + bt_ref[...]
    o_ref[...] = y + x_ref[...]


def _k4_dense(hp, xs, degp2, x, wt, b, gamma, beta):
    return pl.pallas_call(
        _k4_body,
        grid=(GRID,),
        in_specs=[
            pl.BlockSpec((R, D), lambda i: (i, 0)),
            pl.BlockSpec((R, D), lambda i: (i, 0)),
            pl.BlockSpec((NC, R, 1), lambda i: (0, i, 0)),
            pl.BlockSpec((R, D), lambda i: (i, 0)),
            pl.BlockSpec((D, D), lambda i: (0, 0)),
            pl.BlockSpec((1, D), lambda i: (0, 0)),
            pl.BlockSpec((1, D), lambda i: (0, 0)),
            pl.BlockSpec((1, D), lambda i: (0, 0)),
        ],
        out_specs=pl.BlockSpec((R, D), lambda i: (i, 0)),
        out_shape=jax.ShapeDtypeStruct((N, D), jnp.float32),
    )(hp, xs, degp2, x, wt, b, gamma, beta)


def kernel(x, edge_index, W, b, gamma, beta):
    row = edge_index[0]
    col = edge_index[1]
    pad = EP - E
    trash = N + (jnp.arange(pad, dtype=jnp.int32) % (NP - N))
    row3 = jnp.concatenate([row, trash]).reshape(NBT, B)
    col3 = jnp.concatenate([col, jnp.zeros((pad,), jnp.int32)]).reshape(NBT, B)

    degp = _k1_deg()(row3)
    degp2 = degp.reshape(NC, NP, 1)
    xs = _k2_prescale(degp2, x)
    hp = _k3_spmm()(col3, row3, xs)
    return _k4_dense(
        hp, xs, degp2, x, W.T, b.reshape(1, D), gamma.reshape(1, D), beta.reshape(1, D)
    )


# speedup vs baseline: 1.1884x; 1.1884x over previous
import functools

import jax
import jax.numpy as jnp
from jax import lax
from jax.experimental import pallas as pl
from jax.experimental.pallas import tpu as pltpu
from jax.experimental.pallas import tpu_sc as plsc

N = 10000
E = 320000
D = 128

NC = 2
NS = 16
B = 128

NB = 80
NBT = NC * NS * NB
CH = 8
NB0 = 120
NB1 = 2 * NB - NB0
EWK = NB * B
EP = NC * NS * EWK
NP = 10112
RPS = NP // NS

def _make_mesh():
    return plsc.VectorSubcoreMesh(
        core_axis_name="c", subcore_axis_name="s", num_cores=NC, num_subcores=NS
    )


def _k1_body(row3_hbm, deg_out, deg_sp, idx_all, ones_v, zero_v, sem):
    cid = lax.axis_index("c")
    sid = lax.axis_index("s")
    w = cid * NS + sid

    for i in range(B // 16):
        ones_v[pl.ds(i * 16, 16)] = jnp.full((16,), 1.0, jnp.float32)
    for i in range(RPS // 16):
        zero_v[pl.ds(i * 16, 16)] = jnp.zeros((16,), jnp.float32)
    pltpu.sync_copy(zero_v, deg_sp.at[pl.ds(sid * RPS, RPS)])
    pltpu.sync_copy(row3_hbm.at[pl.ds(w * NB, NB)], idx_all)
    plsc.subcore_barrier()

    def body(j, carry):
        pltpu.sync_copy(ones_v, deg_sp.at[idx_all.at[j]], add=True)
        return carry

    lax.fori_loop(0, NB, body, 0)
    plsc.subcore_barrier()
    pltpu.sync_copy(deg_sp.at[pl.ds(sid * RPS, RPS)], zero_v)
    pltpu.sync_copy(zero_v, deg_out.at[pl.ds(cid * NP + sid * RPS, RPS)])


@functools.cache
def _k1_deg():
    return pl.kernel(
        _k1_body,
        mesh=_make_mesh(),
        out_type=jax.ShapeDtypeStruct((NC * NP,), jnp.float32),
        scratch_types=[
            pltpu.VMEM_SHARED((NP,), jnp.float32),
            pltpu.VMEM((NB, B), jnp.int32),
            pltpu.VMEM((B,), jnp.float32),
            pltpu.VMEM((RPS,), jnp.float32),
            pltpu.SemaphoreType.DMA,
        ],
    )


def _k3_body(col3_hbm, row3_hbm, xs_hbm, h_out, h_sp, idx_c, idx_r, msg0, msg1, sg0, sg1):
    cid = lax.axis_index("c")
    sid = lax.axis_index("s")
    w = cid * NS + sid

    def zrow(r, carry):
        for i in range(D // 16):
            msg0[r, pl.ds(i * 16, 16)] = jnp.zeros((16,), jnp.float32)
        return carry

    lax.fori_loop(0, B, zrow, 0)
    r0 = sid * RPS
    rem = RPS % B
    for k in range(RPS // B):
        pltpu.sync_copy(msg0, h_sp.at[pl.ds(r0 + k * B, B), :])
    pltpu.sync_copy(
        msg0.at[pl.ds(0, rem), :], h_sp.at[pl.ds(r0 + (RPS // B) * B, rem), :]
    )
    plsc.subcore_barrier()

    bt0 = lax.select(cid == 0, sid * NB0, NS * NB0 + sid * NB1)
    nch = lax.select(cid == 0, NB0 // CH, NB1 // CH)

    def chunk(c, carry):
        pltpu.sync_copy(col3_hbm.at[pl.ds(bt0 + c * CH, CH)], idx_c)
        pltpu.sync_copy(row3_hbm.at[pl.ds(bt0 + c * CH, CH)], idx_r)
        bufs = (msg0, msg1)
        sems = (sg0, sg1)
        descs = [None] * CH
        descs[0] = pltpu.async_copy(xs_hbm.at[idx_c.at[0]], msg0, sg0)
        for j in range(CH):
            if j + 1 < CH:
                descs[j + 1] = pltpu.async_copy(
                    xs_hbm.at[idx_c.at[j + 1]], bufs[(j + 1) % 2], sems[(j + 1) % 2]
                )
            descs[j].wait()
            pltpu.sync_copy(bufs[j % 2], h_sp.at[idx_r.at[j]], add=True)
        return carry

    lax.fori_loop(0, nch, chunk, 0)
    plsc.subcore_barrier()
    for k in range(RPS // B):
        pltpu.sync_copy(h_sp.at[pl.ds(r0 + k * B, B), :], msg0)
        pltpu.sync_copy(msg0, h_out.at[cid, pl.ds(r0 + k * B, B), :])
    pltpu.sync_copy(
        h_sp.at[pl.ds(r0 + (RPS // B) * B, rem), :], msg0.at[pl.ds(0, rem), :]
    )
    pltpu.sync_copy(
        msg0.at[pl.ds(0, rem), :], h_out.at[cid, pl.ds(r0 + (RPS // B) * B, rem), :]
    )


@functools.cache
def _k3_spmm():
    return pl.kernel(
        _k3_body,
        mesh=_make_mesh(),
        out_type=jax.ShapeDtypeStruct((NC, NP, D), jnp.float32),
        scratch_types=[
            pltpu.VMEM_SHARED((NP, D), jnp.float32),
            pltpu.VMEM((CH, B), jnp.int32),
            pltpu.VMEM((CH, B), jnp.int32),
            pltpu.VMEM((B, D), jnp.float32),
            pltpu.VMEM((B, D), jnp.float32),
            pltpu.SemaphoreType.DMA,
            pltpu.SemaphoreType.DMA,
        ],
    )


R = 1000
GRID = N // R


def _k2_body(dp_ref, x_ref, xs_ref):
    d = dp_ref[0] + dp_ref[1] + 1.0
    dis = lax.rsqrt(d + 1e-12)
    xs_ref[...] = x_ref[...] * dis


def _k2_prescale(degp2, x):
    return pl.pallas_call(
        _k2_body,
        grid=(GRID,),
        in_specs=[
            pl.BlockSpec((NC, R, 1), lambda i: (0, i, 0)),
            pl.BlockSpec((R, D), lambda i: (i, 0)),
        ],
        out_specs=pl.BlockSpec((R, D), lambda i: (i, 0)),
        out_shape=jax.ShapeDtypeStruct((N, D), jnp.float32),
    )(degp2, x)


def _k4_body(ta_ref, tb_ref, xs_ref, dp_ref, x_ref, wt_ref, b_ref, g_ref, bt_ref, o_ref):
    d = dp_ref[0] + dp_ref[1] + 1.0
    dis = lax.rsqrt(d + 1e-12)
    h = (ta_ref[0] + tb_ref[0] + xs_ref[...]) * dis
    y = jnp.dot(h, wt_ref[...], preferred_element_type=jnp.float32) + b_ref[...]
    y = 0.5 * y * (1.0 + lax.erf(y * 0.7071067811865475))
    mu = jnp.mean(y, axis=-1, keepdims=True)
    yc = y - mu
    var = jnp.mean(yc * yc, axis=-1, keepdims=True)
    y = yc * lax.rsqrt(var + 1e-5) * g_ref[...] + bt_ref[...]
    o_ref[...] = y + x_ref[...]


def _k4_dense(hp, xs, degp2, x, wt, b, gamma, beta):
    return pl.pallas_call(
        _k4_body,
        grid=(GRID,),
        in_specs=[
            pl.BlockSpec((1, R, D), lambda i: (0, i, 0)),
            pl.BlockSpec((1, R, D), lambda i: (1, i, 0)),
            pl.BlockSpec((R, D), lambda i: (i, 0)),
            pl.BlockSpec((NC, R, 1), lambda i: (0, i, 0)),
            pl.BlockSpec((R, D), lambda i: (i, 0)),
            pl.BlockSpec((D, D), lambda i: (0, 0)),
            pl.BlockSpec((1, D), lambda i: (0, 0)),
            pl.BlockSpec((1, D), lambda i: (0, 0)),
            pl.BlockSpec((1, D), lambda i: (0, 0)),
        ],
        out_specs=pl.BlockSpec((R, D), lambda i: (i, 0)),
        out_shape=jax.ShapeDtypeStruct((N, D), jnp.float32),
    )(hp, hp, xs, degp2, x, wt, b, gamma, beta)


def kernel(x, edge_index, W, b, gamma, beta):
    row = edge_index[0]
    col = edge_index[1]
    pad = EP - E
    trash = N + (jnp.arange(pad, dtype=jnp.int32) % (NP - N))
    row3 = jnp.concatenate([row, trash]).reshape(NBT, B)
    col3 = jnp.concatenate([col, jnp.zeros((pad,), jnp.int32)]).reshape(NBT, B)

    degp = _k1_deg()(row3)
    degp2 = degp.reshape(NC, NP, 1)
    xs = _k2_prescale(degp2, x)
    hp = _k3_spmm()(col3, row3, xs)
    return _k4_dense(
        hp, xs, degp2, x, W.T, b.reshape(1, D), gamma.reshape(1, D), beta.reshape(1, D)
    )
